# baseline (speedup 1.0000x reference)
import jax
import jax.numpy as jnp
from jax.experimental import pallas as pl
from jax.experimental.pallas import tpu as pltpu

NUM_EMBED = 50
FEATS = 128


def _body(col_t_ref, row_t_ref, o_ref):
    col_t = col_t_ref[...]
    row_t = row_t_ref[...]
    h = row_t.shape[1]
    w = col_t.shape[1]
    o_ref[0, 0:FEATS] = jnp.broadcast_to(col_t[:, None, :], (FEATS, h, w))
    o_ref[0, FEATS : 2 * FEATS] = jnp.broadcast_to(row_t[:, :, None], (FEATS, h, w))


def kernel(mask, row_embed, col_embed):
    B = mask.shape[0]
    h, w = mask.shape[-2], mask.shape[-1]
    d = col_embed.shape[-1]
    col_t = col_embed.T
    row_t = row_embed.T

    out = pl.pallas_call(
        _body,
        grid=(B,),
        in_specs=[
            pl.BlockSpec((d, w), lambda b: (0, 0)),
            pl.BlockSpec((d, h), lambda b: (0, 0)),
        ],
        out_specs=pl.BlockSpec((1, 2 * d, h, w), lambda b: (b, 0, 0, 0)),
        out_shape=jax.ShapeDtypeStruct((B, 2 * d, h, w), jnp.float32),
        compiler_params=pltpu.CompilerParams(
            dimension_semantics=("arbitrary",),
        ),
    )(col_t, row_t)
    return out


# speedup vs baseline: 1.7818x; 1.7818x over previous
import jax
import jax.numpy as jnp
from jax.experimental import pallas as pl
from jax.experimental.pallas import tpu as pltpu

FEATS = 128


def _body(col_t_ref, row_t_ref, o_ref, plane_ref):
    b = pl.program_id(0)

    @pl.when(b == 0)
    def _():
        col_t = col_t_ref[...]
        row_t = row_t_ref[...]
        d, w = col_t.shape
        h = row_t.shape[1]
        plane_ref[0:d] = jnp.broadcast_to(col_t[:, None, :], (d, h, w)).reshape(d, h * w)
        plane_ref[d : 2 * d] = jnp.broadcast_to(row_t[:, :, None], (d, h, w)).reshape(d, h * w)

    o_ref[0] = plane_ref[...]


def kernel(mask, row_embed, col_embed):
    B = mask.shape[0]
    h, w = mask.shape[-2], mask.shape[-1]
    d = col_embed.shape[-1]
    col_t = col_embed.T
    row_t = row_embed.T

    out = pl.pallas_call(
        _body,
        grid=(B,),
        in_specs=[
            pl.BlockSpec((d, w), lambda b: (0, 0)),
            pl.BlockSpec((d, h), lambda b: (0, 0)),
        ],
        out_specs=pl.BlockSpec((1, 2 * d, h * w), lambda b: (b, 0, 0)),
        out_shape=jax.ShapeDtypeStruct((B, 2 * d, h * w), jnp.float32),
        scratch_shapes=[pltpu.VMEM((2 * d, h * w), jnp.float32)],
        compiler_params=pltpu.CompilerParams(
            dimension_semantics=("arbitrary",),
        ),
    )(col_t, row_t)
    return out.reshape(B, 2 * d, h, w)


# speedup vs baseline: 1.7985x; 1.0094x over previous
import jax
import jax.numpy as jnp
from jax.experimental import pallas as pl
from jax.experimental.pallas import tpu as pltpu


def _body(col_t_ref, row_t_ref, o_ref, plane_ref, sems):
    col_t = col_t_ref[...]
    row_t = row_t_ref[...]
    d, w = col_t.shape
    h = row_t.shape[1]
    B = o_ref.shape[0]
    plane_ref[0:d] = jnp.broadcast_to(col_t[:, None, :], (d, h, w)).reshape(d, h * w)
    plane_ref[d : 2 * d] = jnp.broadcast_to(row_t[:, :, None], (d, h, w)).reshape(d, h * w)
    copies = [
        pltpu.make_async_copy(plane_ref, o_ref.at[b], sems.at[b]) for b in range(B)
    ]
    for c in copies:
        c.start()
    for c in copies:
        c.wait()


def kernel(mask, row_embed, col_embed):
    B = mask.shape[0]
    h, w = mask.shape[-2], mask.shape[-1]
    d = col_embed.shape[-1]
    col_t = col_embed.T
    row_t = row_embed.T

    out = pl.pallas_call(
        _body,
        in_specs=[
            pl.BlockSpec(memory_space=pltpu.MemorySpace.VMEM),
            pl.BlockSpec(memory_space=pltpu.MemorySpace.VMEM),
        ],
        out_specs=pl.BlockSpec(memory_space=pl.ANY),
        out_shape=jax.ShapeDtypeStruct((B, 2 * d, h * w), jnp.float32),
        scratch_shapes=[
            pltpu.VMEM((2 * d, h * w), jnp.float32),
            pltpu.SemaphoreType.DMA((B,)),
        ],
    )(col_t, row_t)
    return out.reshape(B, 2 * d, h, w)
